# Initial kernel scaffold; baseline (speedup 1.0000x reference)
import jax
import jax.numpy as jnp
from jax.experimental import pallas as pl


def kernel(x, edge_index, batch, votes, W_in, b_in, W_msg, b_msg, W_out, b_out, W_trunk, b_trunk):
    raise NotImplementedError("write your pallas kernel here")


# speedup vs baseline: 2.7908x; 2.7908x over previous
import functools

import jax
import jax.numpy as jnp
from jax import lax
from jax.experimental import pallas as pl
from jax.experimental.pallas import tpu as pltpu
from jax.experimental.pallas import tpu_sc as plsc

N, E, D, H, B = 10000, 160000, 256, 256, 64
HC = H // 2
NPAD = 10240
EPAD = 163840
BLK = 512
NBLK = NPAD // BLK
NTILE = 16
ROWS_PER_TILE = NPAD // NTILE
EDGES_PER_TILE = EPAD // NTILE
CHUNK = 128
NCHUNK = EDGES_PER_TILE // CHUNK


def _encode_body(x_ref, votes_ref, wx_ref, wv_ref, bin_ref, h0_ref, h1_ref):
    x = x_ref[...]
    v = votes_ref[0, 0, :].reshape(BLK, 1)
    h = jnp.dot(x, wx_ref[...], preferred_element_type=jnp.float32)
    h = h + v * wv_ref[...] + bin_ref[...]
    h = jnp.maximum(h, 0.0)
    h0_ref[...] = h[:, :HC]
    h1_ref[...] = h[:, HC:]


def _encode(xp, votes3, wx, wv, bin2):
    return pl.pallas_call(
        _encode_body,
        grid=(NBLK,),
        in_specs=[
            pl.BlockSpec((BLK, D), lambda i: (i, 0)),
            pl.BlockSpec((1, 1, BLK), lambda i: (i, 0, 0)),
            pl.BlockSpec((D, H), lambda i: (0, 0)),
            pl.BlockSpec((1, H), lambda i: (0, 0)),
            pl.BlockSpec((1, H), lambda i: (0, 0)),
        ],
        out_specs=[
            pl.BlockSpec((BLK, HC), lambda i: (i, 0)),
            pl.BlockSpec((BLK, HC), lambda i: (i, 0)),
        ],
        out_shape=[
            jax.ShapeDtypeStruct((NPAD, HC), jnp.float32),
            jax.ShapeDtypeStruct((NPAD, HC), jnp.float32),
        ],
    )(xp, votes3, wx, wv, bin2)


EDGES_PER_TILE_DEG = EPAD // 2 // NTILE
NCHUNK_DEG = EDGES_PER_TILE_DEG // CHUNK


@functools.cache
def _make_message_pass():
    mesh = plsc.VectorSubcoreMesh(core_axis_name="c", subcore_axis_name="s")

    @functools.partial(
        pl.kernel,
        mesh=mesh,
        out_type=[
            jax.ShapeDtypeStruct((NPAD, HC), jnp.float32),
            jax.ShapeDtypeStruct((NPAD, HC), jnp.float32),
            jax.ShapeDtypeStruct((NPAD, HC), jnp.float32),
            jax.ShapeDtypeStruct((NPAD, HC), jnp.float32),
        ],
        scratch_types=[
            pltpu.VMEM((CHUNK,), jnp.int32),
            pltpu.VMEM((CHUNK,), jnp.int32),
            pltpu.VMEM((CHUNK, HC), jnp.float32),
            pltpu.VMEM((CHUNK, HC), jnp.float32),
            pltpu.VMEM_SHARED((NPAD, HC), jnp.float32),
            pltpu.SemaphoreType.DMA,
        ],
    )
    def mp_body(h0, h1, src, dst, zrows, ones_rows,
                s0_out, s1_out, d0_out, d1_out,
                src_v, dst_v, rows_v, ones_v, acc_s, sem):
        c = lax.axis_index("c")
        s = lax.axis_index("s")
        r0 = s * ROWS_PER_TILE
        pltpu.sync_copy(zrows, acc_s.at[pl.ds(r0, ROWS_PER_TILE)])
        pltpu.sync_copy(ones_rows, ones_v)
        plsc.subcore_barrier()

        def main_loop(h_ref):
            def body(j, carry):
                base = s * EDGES_PER_TILE + j * CHUNK
                pltpu.sync_copy(src.at[pl.ds(base, CHUNK)], src_v)
                pltpu.sync_copy(dst.at[pl.ds(base, CHUNK)], dst_v)
                pltpu.async_copy(h_ref.at[src_v], rows_v, sem).wait()
                pltpu.sync_copy(rows_v, acc_s.at[dst_v], add=True)
                return carry
            lax.fori_loop(0, NCHUNK, body, 0)

        @pl.when(c == 0)
        def _():
            main_loop(h0)

        @pl.when(c == 1)
        def _():
            main_loop(h1)

        plsc.subcore_barrier()

        @pl.when(c == 0)
        def _():
            pltpu.sync_copy(acc_s.at[pl.ds(r0, ROWS_PER_TILE)],
                            s0_out.at[pl.ds(r0, ROWS_PER_TILE)])

        @pl.when(c == 1)
        def _():
            pltpu.sync_copy(acc_s.at[pl.ds(r0, ROWS_PER_TILE)],
                            s1_out.at[pl.ds(r0, ROWS_PER_TILE)])

        plsc.subcore_barrier()

        pltpu.sync_copy(zrows, acc_s.at[pl.ds(r0, ROWS_PER_TILE)])
        plsc.subcore_barrier()

        def deg_loop(base_core):
            def body(j, carry):
                base = base_core + s * EDGES_PER_TILE_DEG + j * CHUNK
                pltpu.sync_copy(dst.at[pl.ds(base, CHUNK)], dst_v)
                pltpu.sync_copy(ones_v, acc_s.at[dst_v], add=True)
                return carry
            lax.fori_loop(0, NCHUNK_DEG, body, 0)

        @pl.when(c == 0)
        def _():
            deg_loop(0)

        @pl.when(c == 1)
        def _():
            deg_loop(EPAD // 2)

        plsc.subcore_barrier()

        @pl.when(c == 0)
        def _():
            pltpu.sync_copy(acc_s.at[pl.ds(r0, ROWS_PER_TILE)],
                            d0_out.at[pl.ds(r0, ROWS_PER_TILE)])

        @pl.when(c == 1)
        def _():
            pltpu.sync_copy(acc_s.at[pl.ds(r0, ROWS_PER_TILE)],
                            d1_out.at[pl.ds(r0, ROWS_PER_TILE)])

    return mp_body


def _message_pass(h0, h1, src, dst, zrows, ones_rows):
    return _make_message_pass()(h0, h1, src, dst, zrows, ones_rows)


def _final_body(h0_ref, h1_ref, s0_ref, s1_ref, d0_ref, d1_ref, batch_ref,
                wmsg_ref, bmsg_ref, wout_ref, bout_ref, wtrunk_ref, btrunk_ref,
                out_ref, pooled_acc, cnt_acc):
    i = pl.program_id(0)

    @pl.when(i == 0)
    def _():
        pooled_acc[...] = jnp.zeros_like(pooled_acc)
        cnt_acc[...] = jnp.zeros_like(cnt_acc)

    h = jnp.concatenate([h0_ref[...], h1_ref[...]], axis=1)
    sums = jnp.concatenate([s0_ref[...], s1_ref[...]], axis=1)
    deg = (d0_ref[...] + d1_ref[...])[:, 0:1]
    agg = jnp.dot(sums, wmsg_ref[...], preferred_element_type=jnp.float32)
    agg = (agg + deg * bmsg_ref[...]) / jnp.maximum(deg, 1.0)
    h2 = jnp.maximum(h + agg, 0.0)

    b = batch_ref[0, 0, :]
    onehot = (b[None, :] == lax.broadcasted_iota(jnp.int32, (B, BLK), 0)
              ).astype(jnp.float32)
    pooled_acc[...] += jnp.dot(onehot, h2, preferred_element_type=jnp.float32)
    cnt_acc[...] += jnp.sum(onehot, axis=1, keepdims=True)

    @pl.when(i == NBLK - 1)
    def _():
        cnt = jnp.max(cnt_acc[...], axis=1, keepdims=True)
        pooled = pooled_acc[...] / jnp.maximum(cnt, 1.0)
        enc = jnp.dot(pooled, wout_ref[...], preferred_element_type=jnp.float32)
        enc = jnp.maximum(enc + bout_ref[...], 0.0)
        out_ref[...] = (jnp.dot(enc, wtrunk_ref[...],
                                preferred_element_type=jnp.float32)
                        + btrunk_ref[...])


def _final(h0, h1, s0, s1, d0, d1, batch3, wmsg, bmsg2, wout, bout2, wtrunkp, btrunk2):
    return pl.pallas_call(
        _final_body,
        grid=(NBLK,),
        in_specs=[
            pl.BlockSpec((BLK, HC), lambda i: (i, 0)),
            pl.BlockSpec((BLK, HC), lambda i: (i, 0)),
            pl.BlockSpec((BLK, HC), lambda i: (i, 0)),
            pl.BlockSpec((BLK, HC), lambda i: (i, 0)),
            pl.BlockSpec((BLK, HC), lambda i: (i, 0)),
            pl.BlockSpec((BLK, HC), lambda i: (i, 0)),
            pl.BlockSpec((1, 1, BLK), lambda i: (i, 0, 0)),
            pl.BlockSpec((H, H), lambda i: (0, 0)),
            pl.BlockSpec((1, H), lambda i: (0, 0)),
            pl.BlockSpec((H, H), lambda i: (0, 0)),
            pl.BlockSpec((1, H), lambda i: (0, 0)),
            pl.BlockSpec((H, 128), lambda i: (0, 0)),
            pl.BlockSpec((1, 128), lambda i: (0, 0)),
        ],
        out_specs=pl.BlockSpec((B, 128), lambda i: (0, 0)),
        out_shape=jax.ShapeDtypeStruct((B, 128), jnp.float32),
        scratch_shapes=[
            pltpu.VMEM((B, H), jnp.float32),
            pltpu.VMEM((B, 128), jnp.float32),
        ],
    )(h0, h1, s0, s1, d0, d1, batch3, wmsg, bmsg2, wout, bout2, wtrunkp, btrunk2)


def kernel(x, edge_index, batch, votes, W_in, b_in, W_msg, b_msg,
           W_out, b_out, W_trunk, b_trunk):
    f32 = jnp.float32
    xp = jnp.pad(x, ((0, NPAD - N), (0, 0)))
    votes3 = jnp.pad(votes[:, 0], (0, NPAD - N)).reshape(NBLK, 1, BLK)
    batch3 = jnp.pad(batch, (0, NPAD - N), constant_values=B).reshape(NBLK, 1, BLK)
    src = jnp.pad(edge_index[0], (0, EPAD - E))
    dst = jnp.pad(edge_index[1], (0, EPAD - E), constant_values=N)

    h0, h1 = _encode(xp, votes3, W_in[:D], W_in[D:D + 1],
                     b_in.reshape(1, H))

    zrows = jnp.zeros((ROWS_PER_TILE, HC), f32)
    ones_rows = jnp.ones((CHUNK, HC), f32)
    s0, s1, d0, d1 = _message_pass(h0, h1, src, dst, zrows, ones_rows)

    wtrunkp = jnp.pad(W_trunk, ((0, 0), (0, 127)))
    out = _final(h0, h1, s0, s1, d0, d1, batch3,
                 W_msg, b_msg.reshape(1, H), W_out, b_out.reshape(1, H),
                 wtrunkp, jnp.pad(b_trunk.reshape(1, 1), ((0, 0), (0, 127))))
    return out[:, :1]
